# baseline (speedup 1.0000x reference)
import functools

import jax
import jax.numpy as jnp
from jax import lax
from jax.experimental import pallas as pl
from jax.experimental.pallas import tpu as pltpu
from jax.experimental.pallas import tpu_sc as plsc

D_MODEL = 1024
N_EXPERTS = 8
TOP_K = 2
D_FF = 4096
TILE_M = 128
ROWS_A = 128

NW = 32


def _router_body(x_ref, g_ref, b_ref, wr_ref, normed_ref, i0_ref, i1_ref,
                 w0_ref, w1_ref, aux_ref, psum, asum):
    i = pl.program_id(0)
    n = pl.num_programs(0)
    xt = x_ref[...]
    m = jnp.mean(xt, axis=-1, keepdims=True)
    v = jnp.mean((xt - m) ** 2, axis=-1, keepdims=True)
    normed = (xt - m) * lax.rsqrt(v + 1e-5) * g_ref[...] + b_ref[...]
    normed_ref[...] = normed
    logits = jnp.dot(normed, wr_ref[...], preferred_element_type=jnp.float32)
    logits = logits - jnp.max(logits, axis=-1, keepdims=True)
    el = jnp.exp(logits)
    probs = el / jnp.sum(el, axis=-1, keepdims=True)
    m1 = jnp.max(probs, axis=-1, keepdims=True)
    i1 = jnp.argmax(probs, axis=-1).astype(jnp.int32)
    eids = lax.broadcasted_iota(jnp.int32, probs.shape, 1)
    masked = jnp.where(eids == i1[:, None], -jnp.inf, probs)
    m2 = jnp.max(masked, axis=-1, keepdims=True)
    i2 = jnp.argmax(masked, axis=-1).astype(jnp.int32)
    s = m1 + m2
    w0 = (m1 / s)[:, 0]
    w1 = (m2 / s)[:, 0]
    i0_ref[...] = i1.reshape(1, 1, ROWS_A)
    i1_ref[...] = i2.reshape(1, 1, ROWS_A)
    w0_ref[...] = w0.reshape(1, 1, ROWS_A)
    w1_ref[...] = w1.reshape(1, 1, ROWS_A)

    @pl.when(i == 0)
    def _init():
        psum[...] = jnp.zeros_like(psum)
        asum[...] = jnp.zeros_like(asum)

    psum[...] += jnp.sum(probs, axis=0, keepdims=True)
    cnt = (jnp.sum((eids == i1[:, None]).astype(jnp.float32), axis=0)
           + jnp.sum((eids == i2[:, None]).astype(jnp.float32), axis=0))
    asum[...] += cnt[None, :]

    @pl.when(i == n - 1)
    def _fin():
        t_total = jnp.float32(n * ROWS_A)
        f = asum[...] / (t_total * TOP_K)
        p_mean = psum[...] / t_total
        aux_ref[...] = jnp.float32(N_EXPERTS) * jnp.sum(f * p_mean).reshape(1, 1)


def _run_router(flat_x, gamma, beta, wr):
    t = flat_x.shape[0]
    nt = t // ROWS_A
    return pl.pallas_call(
        _router_body,
        grid=(nt,),
        in_specs=[
            pl.BlockSpec((ROWS_A, D_MODEL), lambda i: (i, 0)),
            pl.BlockSpec((1, D_MODEL), lambda i: (0, 0)),
            pl.BlockSpec((1, D_MODEL), lambda i: (0, 0)),
            pl.BlockSpec((D_MODEL, N_EXPERTS), lambda i: (0, 0)),
        ],
        out_specs=[
            pl.BlockSpec((ROWS_A, D_MODEL), lambda i: (i, 0)),
            pl.BlockSpec((1, 1, ROWS_A), lambda i: (i, 0, 0)),
            pl.BlockSpec((1, 1, ROWS_A), lambda i: (i, 0, 0)),
            pl.BlockSpec((1, 1, ROWS_A), lambda i: (i, 0, 0)),
            pl.BlockSpec((1, 1, ROWS_A), lambda i: (i, 0, 0)),
            pl.BlockSpec((1, 1), lambda i: (0, 0)),
        ],
        out_shape=[
            jax.ShapeDtypeStruct((t, D_MODEL), jnp.float32),
            jax.ShapeDtypeStruct((nt, 1, ROWS_A), jnp.int32),
            jax.ShapeDtypeStruct((nt, 1, ROWS_A), jnp.int32),
            jax.ShapeDtypeStruct((nt, 1, ROWS_A), jnp.float32),
            jax.ShapeDtypeStruct((nt, 1, ROWS_A), jnp.float32),
            jax.ShapeDtypeStruct((1, 1), jnp.float32),
        ],
        scratch_shapes=[
            pltpu.VMEM((1, N_EXPERTS), jnp.float32),
            pltpu.VMEM((1, N_EXPERTS), jnp.float32),
        ],
        compiler_params=pltpu.CompilerParams(
            dimension_semantics=("arbitrary",)),
    )(flat_x, gamma.reshape(1, -1), beta.reshape(1, -1), wr)


def _make_gather(n_rows):
    per_w = n_rows // NW
    ch = 48 if per_w % 48 == 0 else 32
    nch = per_w // ch
    mesh = plsc.VectorSubcoreMesh(core_axis_name="c", subcore_axis_name="s")

    @functools.partial(
        pl.kernel, mesh=mesh,
        out_type=jax.ShapeDtypeStruct((n_rows, D_MODEL), jnp.float32),
        scratch_types=[
            pltpu.VMEM((per_w,), jnp.int32),
            pltpu.VMEM((ch, D_MODEL), jnp.float32),
            pltpu.VMEM((ch, D_MODEL), jnp.float32),
            pltpu.SemaphoreType.DMA,
            pltpu.SemaphoreType.DMA,
        ],
    )
    def gather_k(table_hbm, idx_hbm, out_hbm, idx_v, buf0, buf1, sem0, sem1):
        wid = lax.axis_index("s") * 2 + lax.axis_index("c")
        base = wid * per_w
        pltpu.sync_copy(idx_hbm.at[pl.ds(base, per_w)], idx_v)
        bufs = (buf0, buf1)
        sems = (sem0, sem1)
        cps = [None, None]
        for i in range(nch):
            k = i & 1
            if cps[k] is not None:
                cps[k].wait()
                pltpu.sync_copy(bufs[k],
                                out_hbm.at[pl.ds(base + (i - 2) * ch, ch)])
            cps[k] = pltpu.async_copy(
                table_hbm.at[idx_v.at[pl.ds(i * ch, ch)]], bufs[k], sems[k])
        for i in range(max(nch - 2, 0), nch):
            k = i & 1
            cps[k].wait()
            pltpu.sync_copy(bufs[k], out_hbm.at[pl.ds(base + i * ch, ch)])

    return gather_k


def _ffn_body(te_ref, xs_ref, w1_ref, b1_ref, w2_ref, b2_ref, wrow_ref,
              ys_ref, acc):
    xt = xs_ref[...].astype(jnp.bfloat16)
    ft = 512
    for j in range(D_FF // ft):
        sl = slice(j * ft, (j + 1) * ft)
        hj = jnp.dot(xt, w1_ref[0, :, sl], preferred_element_type=jnp.float32)
        hj = jax.nn.gelu(hj + b1_ref[0, 0, sl][None, :])
        hjb = hj.astype(jnp.bfloat16)
        for nblk in range(2):
            nsl = slice(nblk * 512, (nblk + 1) * 512)
            pj = jnp.dot(hjb, w2_ref[0, sl, nsl],
                         preferred_element_type=jnp.float32)
            if j == 0:
                acc[:, nsl] = pj
            else:
                acc[:, nsl] += pj
    y = (acc[...] + b2_ref[0, 0][None, :]) * wrow_ref[...]
    ys_ref[...] = y


def _run_ffn(xs, w1b, b1, w2b, b2, w_pad, te):
    p_max = xs.shape[0]
    n_m = p_max // TILE_M
    grid_spec = pltpu.PrefetchScalarGridSpec(
        num_scalar_prefetch=1,
        grid=(n_m,),
        in_specs=[
            pl.BlockSpec((TILE_M, D_MODEL), lambda i, te: (i, 0)),
            pl.BlockSpec((1, D_MODEL, D_FF), lambda i, te: (te[i], 0, 0)),
            pl.BlockSpec((1, 1, D_FF), lambda i, te: (te[i], 0, 0)),
            pl.BlockSpec((1, D_FF, D_MODEL), lambda i, te: (te[i], 0, 0)),
            pl.BlockSpec((1, 1, D_MODEL), lambda i, te: (te[i], 0, 0)),
            pl.BlockSpec((TILE_M, 1), lambda i, te: (i, 0)),
        ],
        out_specs=pl.BlockSpec((TILE_M, D_MODEL), lambda i, te: (i, 0)),
        scratch_shapes=[pltpu.VMEM((TILE_M, D_MODEL), jnp.float32)],
    )
    return pl.pallas_call(
        _ffn_body,
        grid_spec=grid_spec,
        out_shape=jax.ShapeDtypeStruct((p_max, D_MODEL), jnp.float32),
        compiler_params=pltpu.CompilerParams(
            dimension_semantics=("arbitrary",)),
    )(te, xs, w1b, b1.reshape(N_EXPERTS, 1, D_FF), w2b,
      b2.reshape(N_EXPERTS, 1, D_MODEL), w_pad.reshape(p_max, 1))


def _add_body(x_ref, yp_ref, o_ref):
    o_ref[...] = (x_ref[...] + yp_ref[:, :D_MODEL] + yp_ref[:, D_MODEL:])


def _run_add(flat_x, yp2):
    t = flat_x.shape[0]
    return pl.pallas_call(
        _add_body,
        grid=(t // ROWS_A,),
        in_specs=[
            pl.BlockSpec((ROWS_A, D_MODEL), lambda i: (i, 0)),
            pl.BlockSpec((ROWS_A, 2 * D_MODEL), lambda i: (i, 0)),
        ],
        out_specs=pl.BlockSpec((ROWS_A, D_MODEL), lambda i: (i, 0)),
        out_shape=jax.ShapeDtypeStruct((t, D_MODEL), jnp.float32),
        compiler_params=pltpu.CompilerParams(
            dimension_semantics=("arbitrary",)),
    )(flat_x, yp2)


def kernel(x, gamma, beta, Wr, W1, b1, W2, b2):
    b, s, d = x.shape
    t = b * s
    flat_x = x.reshape(t, d)

    normed, i0, i1, w0, w1, aux = _run_router(flat_x, gamma, beta, Wr)
    aux_loss = aux[0, 0]

    i0f = i0.reshape(t)
    i1f = i1.reshape(t)
    w0f = w0.reshape(t)
    w1f = w1.reshape(t)
    flat_e = jnp.stack([i0f, i1f], axis=1).reshape(2 * t)
    flat_w = jnp.stack([w0f, w1f], axis=1).reshape(2 * t)
    flat_t = jnp.repeat(jnp.arange(t, dtype=jnp.int32), 2)
    oh = (flat_e[:, None] == jnp.arange(N_EXPERTS, dtype=jnp.int32)[None, :])
    oh = oh.astype(jnp.int32)
    csum = jnp.cumsum(oh, axis=0)
    rank = jnp.take_along_axis(csum, flat_e[:, None], axis=1)[:, 0] - 1
    counts = csum[-1]
    pc = ((counts + TILE_M - 1) // TILE_M) * TILE_M
    ends = jnp.cumsum(pc)
    starts = ends - pc
    p_max = 2 * t + N_EXPERTS * TILE_M
    n_m = p_max // TILE_M
    pos = starts[flat_e] + rank
    src_pad = jnp.zeros((p_max,), jnp.int32).at[pos].set(flat_t)
    w_pad = jnp.zeros((p_max,), jnp.float32).at[pos].set(flat_w)
    tile_expert = jnp.searchsorted(
        ends, jnp.arange(n_m, dtype=jnp.int32) * TILE_M, side="right"
    ).astype(jnp.int32)

    xs = _make_gather(p_max)(normed, src_pad)

    w1b = W1.astype(jnp.bfloat16)
    w2b = W2.astype(jnp.bfloat16)
    ys = _run_ffn(xs, w1b, b1, w2b, b2, w_pad, tile_expert)

    yp = _make_gather(2 * t)(ys, pos.astype(jnp.int32))
    out_flat = _run_add(flat_x, yp.reshape(t, 2 * D_MODEL))
    return out_flat.reshape(b, s, d), aux_loss


# speedup vs baseline: 1.0284x; 1.0284x over previous
import functools

import jax
import jax.numpy as jnp
from jax import lax
from jax.experimental import pallas as pl
from jax.experimental.pallas import tpu as pltpu
from jax.experimental.pallas import tpu_sc as plsc

D_MODEL = 1024
N_EXPERTS = 8
TOP_K = 2
D_FF = 4096
TILE_M = 256
ROWS_A = 128

NW = 32


def _router_body(x_ref, g_ref, b_ref, wr_ref, normed_ref, i0_ref, i1_ref,
                 w0_ref, w1_ref, aux_ref, psum, asum):
    i = pl.program_id(0)
    n = pl.num_programs(0)
    xt = x_ref[...]
    m = jnp.mean(xt, axis=-1, keepdims=True)
    v = jnp.mean((xt - m) ** 2, axis=-1, keepdims=True)
    normed = (xt - m) * lax.rsqrt(v + 1e-5) * g_ref[...] + b_ref[...]
    normed_ref[...] = normed
    logits = jnp.dot(normed, wr_ref[...], preferred_element_type=jnp.float32)
    logits = logits - jnp.max(logits, axis=-1, keepdims=True)
    el = jnp.exp(logits)
    probs = el / jnp.sum(el, axis=-1, keepdims=True)
    m1 = jnp.max(probs, axis=-1, keepdims=True)
    i1 = jnp.argmax(probs, axis=-1).astype(jnp.int32)
    eids = lax.broadcasted_iota(jnp.int32, probs.shape, 1)
    masked = jnp.where(eids == i1[:, None], -jnp.inf, probs)
    m2 = jnp.max(masked, axis=-1, keepdims=True)
    i2 = jnp.argmax(masked, axis=-1).astype(jnp.int32)
    s = m1 + m2
    w0 = (m1 / s)[:, 0]
    w1 = (m2 / s)[:, 0]
    i0_ref[...] = i1.reshape(1, 1, ROWS_A)
    i1_ref[...] = i2.reshape(1, 1, ROWS_A)
    w0_ref[...] = w0.reshape(1, 1, ROWS_A)
    w1_ref[...] = w1.reshape(1, 1, ROWS_A)

    @pl.when(i == 0)
    def _init():
        psum[...] = jnp.zeros_like(psum)
        asum[...] = jnp.zeros_like(asum)

    psum[...] += jnp.sum(probs, axis=0, keepdims=True)
    cnt = (jnp.sum((eids == i1[:, None]).astype(jnp.float32), axis=0)
           + jnp.sum((eids == i2[:, None]).astype(jnp.float32), axis=0))
    asum[...] += cnt[None, :]

    @pl.when(i == n - 1)
    def _fin():
        t_total = jnp.float32(n * ROWS_A)
        f = asum[...] / (t_total * TOP_K)
        p_mean = psum[...] / t_total
        aux_ref[...] = jnp.float32(N_EXPERTS) * jnp.sum(f * p_mean).reshape(1, 1)


def _run_router(flat_x, gamma, beta, wr):
    t = flat_x.shape[0]
    nt = t // ROWS_A
    return pl.pallas_call(
        _router_body,
        grid=(nt,),
        in_specs=[
            pl.BlockSpec((ROWS_A, D_MODEL), lambda i: (i, 0)),
            pl.BlockSpec((1, D_MODEL), lambda i: (0, 0)),
            pl.BlockSpec((1, D_MODEL), lambda i: (0, 0)),
            pl.BlockSpec((D_MODEL, N_EXPERTS), lambda i: (0, 0)),
        ],
        out_specs=[
            pl.BlockSpec((ROWS_A, D_MODEL), lambda i: (i, 0)),
            pl.BlockSpec((1, 1, ROWS_A), lambda i: (i, 0, 0)),
            pl.BlockSpec((1, 1, ROWS_A), lambda i: (i, 0, 0)),
            pl.BlockSpec((1, 1, ROWS_A), lambda i: (i, 0, 0)),
            pl.BlockSpec((1, 1, ROWS_A), lambda i: (i, 0, 0)),
            pl.BlockSpec((1, 1), lambda i: (0, 0)),
        ],
        out_shape=[
            jax.ShapeDtypeStruct((t, D_MODEL), jnp.float32),
            jax.ShapeDtypeStruct((nt, 1, ROWS_A), jnp.int32),
            jax.ShapeDtypeStruct((nt, 1, ROWS_A), jnp.int32),
            jax.ShapeDtypeStruct((nt, 1, ROWS_A), jnp.float32),
            jax.ShapeDtypeStruct((nt, 1, ROWS_A), jnp.float32),
            jax.ShapeDtypeStruct((1, 1), jnp.float32),
        ],
        scratch_shapes=[
            pltpu.VMEM((1, N_EXPERTS), jnp.float32),
            pltpu.VMEM((1, N_EXPERTS), jnp.float32),
        ],
        compiler_params=pltpu.CompilerParams(
            dimension_semantics=("arbitrary",)),
    )(flat_x, gamma.reshape(1, -1), beta.reshape(1, -1), wr)


def _make_gather(n_rows):
    per_w = n_rows // NW
    ch = 48 if per_w % 48 == 0 else 32
    nch = per_w // ch
    mesh = plsc.VectorSubcoreMesh(core_axis_name="c", subcore_axis_name="s")

    @functools.partial(
        pl.kernel, mesh=mesh,
        out_type=jax.ShapeDtypeStruct((n_rows, D_MODEL), jnp.float32),
        scratch_types=[
            pltpu.VMEM((per_w,), jnp.int32),
            pltpu.VMEM((ch, D_MODEL), jnp.float32),
            pltpu.VMEM((ch, D_MODEL), jnp.float32),
            pltpu.SemaphoreType.DMA,
            pltpu.SemaphoreType.DMA,
        ],
    )
    def gather_k(table_hbm, idx_hbm, out_hbm, idx_v, buf0, buf1, sem0, sem1):
        wid = lax.axis_index("s") * 2 + lax.axis_index("c")
        base = wid * per_w
        pltpu.sync_copy(idx_hbm.at[pl.ds(base, per_w)], idx_v)
        bufs = (buf0, buf1)
        sems = (sem0, sem1)
        cps = [None, None]
        for i in range(nch):
            k = i & 1
            if cps[k] is not None:
                cps[k].wait()
                pltpu.sync_copy(bufs[k],
                                out_hbm.at[pl.ds(base + (i - 2) * ch, ch)])
            cps[k] = pltpu.async_copy(
                table_hbm.at[idx_v.at[pl.ds(i * ch, ch)]], bufs[k], sems[k])
        for i in range(max(nch - 2, 0), nch):
            k = i & 1
            cps[k].wait()
            pltpu.sync_copy(bufs[k], out_hbm.at[pl.ds(base + i * ch, ch)])

    return gather_k


def _ffn_body(te_ref, xs_ref, w1_ref, b1_ref, w2_ref, b2_ref, wrow_ref,
              ys_ref, acc):
    xt = xs_ref[...].astype(jnp.bfloat16)
    ft = 512
    for j in range(D_FF // ft):
        sl = slice(j * ft, (j + 1) * ft)
        hj = jnp.dot(xt, w1_ref[0, :, sl], preferred_element_type=jnp.float32)
        hj = jax.nn.gelu(hj + b1_ref[0, 0, sl][None, :])
        hjb = hj.astype(jnp.bfloat16)
        for nblk in range(2):
            nsl = slice(nblk * 512, (nblk + 1) * 512)
            pj = jnp.dot(hjb, w2_ref[0, sl, nsl],
                         preferred_element_type=jnp.float32)
            if j == 0:
                acc[:, nsl] = pj
            else:
                acc[:, nsl] += pj
    y = (acc[...] + b2_ref[0, 0][None, :]) * wrow_ref[...]
    ys_ref[...] = y


def _run_ffn(xs, w1b, b1, w2b, b2, w_pad, te):
    p_max = xs.shape[0]
    n_m = p_max // TILE_M
    grid_spec = pltpu.PrefetchScalarGridSpec(
        num_scalar_prefetch=1,
        grid=(n_m,),
        in_specs=[
            pl.BlockSpec((TILE_M, D_MODEL), lambda i, te: (i, 0)),
            pl.BlockSpec((1, D_MODEL, D_FF), lambda i, te: (te[i], 0, 0)),
            pl.BlockSpec((1, 1, D_FF), lambda i, te: (te[i], 0, 0)),
            pl.BlockSpec((1, D_FF, D_MODEL), lambda i, te: (te[i], 0, 0)),
            pl.BlockSpec((1, 1, D_MODEL), lambda i, te: (te[i], 0, 0)),
            pl.BlockSpec((TILE_M, 1), lambda i, te: (i, 0)),
        ],
        out_specs=pl.BlockSpec((TILE_M, D_MODEL), lambda i, te: (i, 0)),
        scratch_shapes=[pltpu.VMEM((TILE_M, D_MODEL), jnp.float32)],
    )
    return pl.pallas_call(
        _ffn_body,
        grid_spec=grid_spec,
        out_shape=jax.ShapeDtypeStruct((p_max, D_MODEL), jnp.float32),
        compiler_params=pltpu.CompilerParams(
            dimension_semantics=("arbitrary",)),
    )(te, xs, w1b, b1.reshape(N_EXPERTS, 1, D_FF), w2b,
      b2.reshape(N_EXPERTS, 1, D_MODEL), w_pad.reshape(p_max, 1))


def _add_body(x_ref, yp_ref, o_ref):
    o_ref[...] = (x_ref[...] + yp_ref[:, :D_MODEL] + yp_ref[:, D_MODEL:])


def _run_add(flat_x, yp2):
    t = flat_x.shape[0]
    return pl.pallas_call(
        _add_body,
        grid=(t // ROWS_A,),
        in_specs=[
            pl.BlockSpec((ROWS_A, D_MODEL), lambda i: (i, 0)),
            pl.BlockSpec((ROWS_A, 2 * D_MODEL), lambda i: (i, 0)),
        ],
        out_specs=pl.BlockSpec((ROWS_A, D_MODEL), lambda i: (i, 0)),
        out_shape=jax.ShapeDtypeStruct((t, D_MODEL), jnp.float32),
        compiler_params=pltpu.CompilerParams(
            dimension_semantics=("arbitrary",)),
    )(flat_x, yp2)


def kernel(x, gamma, beta, Wr, W1, b1, W2, b2):
    b, s, d = x.shape
    t = b * s
    flat_x = x.reshape(t, d)

    normed, i0, i1, w0, w1, aux = _run_router(flat_x, gamma, beta, Wr)
    aux_loss = aux[0, 0]

    i0f = i0.reshape(t)
    i1f = i1.reshape(t)
    w0f = w0.reshape(t)
    w1f = w1.reshape(t)
    flat_e = jnp.stack([i0f, i1f], axis=1).reshape(2 * t)
    flat_w = jnp.stack([w0f, w1f], axis=1).reshape(2 * t)
    flat_t = jnp.repeat(jnp.arange(t, dtype=jnp.int32), 2)
    oh = (flat_e[:, None] == jnp.arange(N_EXPERTS, dtype=jnp.int32)[None, :])
    oh = oh.astype(jnp.int32)
    csum = jnp.cumsum(oh, axis=0)
    rank = jnp.take_along_axis(csum, flat_e[:, None], axis=1)[:, 0] - 1
    counts = csum[-1]
    pc = ((counts + TILE_M - 1) // TILE_M) * TILE_M
    ends = jnp.cumsum(pc)
    starts = ends - pc
    p_max = 2 * t + N_EXPERTS * TILE_M
    n_m = p_max // TILE_M
    pos = starts[flat_e] + rank
    src_pad = jnp.zeros((p_max,), jnp.int32).at[pos].set(flat_t)
    w_pad = jnp.zeros((p_max,), jnp.float32).at[pos].set(flat_w)
    tile_expert = jnp.searchsorted(
        ends, jnp.arange(n_m, dtype=jnp.int32) * TILE_M, side="right"
    ).astype(jnp.int32)

    xs = _make_gather(p_max)(normed, src_pad)

    w1b = W1.astype(jnp.bfloat16)
    w2b = W2.astype(jnp.bfloat16)
    ys = _run_ffn(xs, w1b, b1, w2b, b2, w_pad, tile_expert)

    yp = _make_gather(2 * t)(ys, pos.astype(jnp.int32))
    out_flat = _run_add(flat_x, yp.reshape(t, 2 * D_MODEL))
    return out_flat.reshape(b, s, d), aux_loss
